# baseline (speedup 1.0000x reference)
import functools

import jax
import jax.numpy as jnp
import numpy as np
from jax import lax
from jax.experimental import pallas as pl
from jax.experimental.pallas import tpu as pltpu
from jax.experimental.pallas import tpu_sc as plsc

_DEPTH = 1000
_D0 = 4096
_COLS = 20
_NC = 2
_NS = 16
_NW = _NC * _NS
_SLABS_PER_W = _D0 // _NW
_BATCH_SLABS = 2
_BATCH_IDX = _BATCH_SLABS * _COLS
_NBATCH = _SLABS_PER_W // _BATCH_SLABS
_IDX_PER_W = _SLABS_PER_W * _COLS
_IDX_PAD = _IDX_PER_W + 16
_NVEC = 3


def _sc_body(idx_hbm, coords_hbm, out_hbm, idx_v, coords_v, buf0, buf1,
             sem0, sem1):
    wid = lax.axis_index("s") * _NC + lax.axis_index("c")
    slab_base = wid * _SLABS_PER_W

    pltpu.async_copy(
        idx_hbm.at[pl.ds(wid * _IDX_PER_W, _IDX_PER_W)],
        idx_v.at[pl.ds(0, _IDX_PER_W)],
        sem0,
    ).wait()
    pltpu.async_copy(coords_hbm, coords_v, sem0).wait()

    zeros16 = jnp.zeros((16,), jnp.float32)
    for buf in (buf0, buf1):
        for s in range(_BATCH_SLABS):
            for c in range(_COLS):
                def zero_step(i, carry, s=s, c=c, buf=buf):
                    buf[s, c, pl.ds(i * 16, 16)] = zeros16
                    return carry

                lax.fori_loop(0, 62, zero_step, 0)
                buf[s, c, pl.ds(_DEPTH - 16, 16)] = zeros16

    ones16 = jnp.ones((16,), jnp.float32)
    mask8 = lax.iota(jnp.int32, 16) < 8

    def scatter_batch(buf, t, vals):
        for j in range(_NVEC):
            s_j = coords_v[pl.ds(j * 16, 16)]
            c_j = coords_v[pl.ds(_NVEC * 16 + j * 16, 16)]
            d = idx_v[pl.ds(t * _BATCH_IDX + j * 16, 16)]
            if j < _NVEC - 1:
                plsc.store_scatter(buf, [s_j, c_j, d], vals)
            else:
                plsc.store_scatter(buf, [s_j, c_j, d], vals, mask=mask8)

    def copy(buf, t, sem):
        return pltpu.make_async_copy(
            buf,
            out_hbm.at[pl.ds(slab_base + t * _BATCH_SLABS, _BATCH_SLABS)],
            sem,
        )

    def pair_step(p, carry):
        for b, buf, sem in ((0, buf0, sem0), (1, buf1, sem1)):
            t = p * 2 + b

            @pl.when(p > 0)
            def _(buf=buf, sem=sem, t=t):
                copy(buf, t - 2, sem).wait()
                scatter_batch(buf, t - 2, zeros16)

            scatter_batch(buf, t, ones16)
            copy(buf, t, sem).start()
        return carry

    lax.fori_loop(0, _NBATCH // 2, pair_step, 0)
    copy(buf0, _NBATCH - 2, sem0).wait()
    copy(buf1, _NBATCH - 1, sem1).wait()


_L = np.arange(_NVEC * 16)
_COORDS = np.concatenate([
    (_L // _COLS) % _BATCH_SLABS,
    _L % _COLS,
]).astype(np.int32)


def kernel(inputs):
    idx = inputs.astype(jnp.int32).reshape(-1)
    coords = jnp.asarray(_COORDS)
    mesh = plsc.VectorSubcoreMesh(core_axis_name="c", subcore_axis_name="s")
    run = functools.partial(
        pl.kernel,
        mesh=mesh,
        compiler_params=pltpu.CompilerParams(needs_layout_passes=False),
        out_type=jax.ShapeDtypeStruct((_D0, _COLS, _DEPTH), jnp.float32),
        scratch_types=[
            pltpu.VMEM((_IDX_PAD,), jnp.int32),
            pltpu.VMEM((2 * _NVEC * 16,), jnp.int32),
            pltpu.VMEM((_BATCH_SLABS, _COLS, _DEPTH), jnp.float32),
            pltpu.VMEM((_BATCH_SLABS, _COLS, _DEPTH), jnp.float32),
            pltpu.SemaphoreType.DMA,
            pltpu.SemaphoreType.DMA,
        ],
    )(_sc_body)
    return run(idx, coords)


# speedup vs baseline: 1.0116x; 1.0116x over previous
import functools

import jax
import jax.numpy as jnp
import numpy as np
from jax import lax
from jax.experimental import pallas as pl
from jax.experimental.pallas import tpu as pltpu
from jax.experimental.pallas import tpu_sc as plsc

_DEPTH = 1000
_D0 = 4096
_COLS = 20
_NC = 2
_NS = 16
_NW = _NC * _NS
_SLABS_PER_W = _D0 // _NW
_BATCH_SLABS = 4
_BATCH_IDX = _BATCH_SLABS * _COLS
_NBATCH = _SLABS_PER_W // _BATCH_SLABS
_IDX_PER_W = _SLABS_PER_W * _COLS


def _sc_body(idx_hbm, coords_hbm, out_hbm, idx_v, coords_v, buf, sem):
    wid = lax.axis_index("s") * _NC + lax.axis_index("c")
    slab_base = wid * _SLABS_PER_W

    pltpu.async_copy(
        idx_hbm.at[pl.ds(wid * _IDX_PER_W, _IDX_PER_W)], idx_v, sem
    ).wait()
    pltpu.async_copy(coords_hbm, coords_v, sem).wait()

    zeros16 = jnp.zeros((16,), jnp.float32)
    for s in range(_BATCH_SLABS):
        for c in range(_COLS):
            def zero_step(i, carry, s=s, c=c):
                buf[s, c, pl.ds(i * 16, 16)] = zeros16
                return carry

            lax.fori_loop(0, 62, zero_step, 0)
            buf[s, c, pl.ds(_DEPTH - 16, 16)] = zeros16

    ones16 = jnp.ones((16,), jnp.float32)

    def batch_step(t, carry):
        for j in range(_BATCH_IDX // 16):
            s_j = coords_v[pl.ds(j * 16, 16)]
            c_j = coords_v[pl.ds(_BATCH_IDX + j * 16, 16)]
            d = idx_v[pl.ds(t * _BATCH_IDX + j * 16, 16)]
            plsc.store_scatter(buf, [s_j, c_j, d], ones16)
        pltpu.sync_copy(
            buf, out_hbm.at[pl.ds(slab_base + t * _BATCH_SLABS, _BATCH_SLABS)]
        )
        for j in range(_BATCH_IDX // 16):
            s_j = coords_v[pl.ds(j * 16, 16)]
            c_j = coords_v[pl.ds(_BATCH_IDX + j * 16, 16)]
            d = idx_v[pl.ds(t * _BATCH_IDX + j * 16, 16)]
            plsc.store_scatter(buf, [s_j, c_j, d], zeros16)
        return carry

    lax.fori_loop(0, _NBATCH, batch_step, 0)


_COORDS = np.concatenate([
    np.arange(_BATCH_IDX) // _COLS,
    np.arange(_BATCH_IDX) % _COLS,
]).astype(np.int32)


def kernel(inputs):
    idx = inputs.astype(jnp.int32).reshape(-1)
    coords = jnp.asarray(_COORDS)
    mesh = plsc.VectorSubcoreMesh(core_axis_name="c", subcore_axis_name="s")
    run = functools.partial(
        pl.kernel,
        mesh=mesh,
        compiler_params=pltpu.CompilerParams(needs_layout_passes=False),
        out_type=jax.ShapeDtypeStruct((_D0, _COLS, _DEPTH), jnp.float32),
        scratch_types=[
            pltpu.VMEM((_IDX_PER_W,), jnp.int32),
            pltpu.VMEM((2 * _BATCH_IDX,), jnp.int32),
            pltpu.VMEM((_BATCH_SLABS, _COLS, _DEPTH), jnp.float32),
            pltpu.SemaphoreType.DMA,
        ],
    )(_sc_body)
    return run(idx, coords)
